# baseline (speedup 1.0000x reference)
import functools

import jax
import jax.numpy as jnp
from jax import lax
from jax.experimental import pallas as pl
from jax.experimental.pallas import tpu as pltpu
from jax.experimental.pallas import tpu_sc as plsc

T = 2048
NHID = 1024
NCLS = 100
CHUNK = 1000
P = 20
PPAD = 32
POUT = 24

NW = 32
H = 50
NSC = NCLS - H
SLOTS = 2


def _sc_gather(x, idx3):
    mesh = plsc.VectorSubcoreMesh(core_axis_name="c", subcore_axis_name="s")

    @functools.partial(
        pl.kernel,
        out_type=jax.ShapeDtypeStruct((NSC, POUT, NHID), jnp.float32),
        mesh=mesh,
        cost_estimate=pl.CostEstimate(
            flops=0, bytes_accessed=16 * 1024 * 1024, transcendentals=0),
        scratch_types=[
            pltpu.VMEM((SLOTS, PPAD), jnp.int32),
            pltpu.VMEM((SLOTS, PPAD, NHID), jnp.float32),
            pltpu.SemaphoreType.DMA,
            pltpu.SemaphoreType.DMA,
            pltpu.SemaphoreType.DMA,
            pltpu.SemaphoreType.DMA,
        ],
    )
    def gather_k(x_hbm, idx_hbm, out_hbm, idx_v, rows_v, g0, g1, w0, w1):
        gsem = (g0, g1)
        wsem = (w0, w1)
        wid = lax.axis_index("s") * 2 + lax.axis_index("c")
        pltpu.sync_copy(idx_hbm.at[:, wid], idx_v)

        def cls_of(k):
            return k * NW + wid

        for k in range(SLOTS):
            @pl.when(cls_of(k) < NSC)
            def _(k=k):
                pltpu.async_copy(x_hbm.at[idx_v.at[k]], rows_v.at[k], gsem[k])
        for k in range(SLOTS):
            @pl.when(cls_of(k) < NSC)
            def _(k=k):
                pltpu.make_async_copy(
                    x_hbm.at[idx_v.at[k]], rows_v.at[k], gsem[k]).wait()
                pltpu.async_copy(rows_v.at[k, pl.ds(0, POUT)],
                                 out_hbm.at[cls_of(k)], wsem[k])
        for k in range(SLOTS):
            @pl.when(cls_of(k) < NSC)
            def _(k=k):
                pltpu.make_async_copy(rows_v.at[k, pl.ds(0, POUT)],
                                      out_hbm.at[cls_of(k)], wsem[k]).wait()

    return gather_k(x, idx3)


def _tcg_body(idx_ref, x_ref, xany_ref, Wc_ref, bc_ref, Ww_ref, bw_ref,
              pclass_ref, pwords_ref, rows_ref, sems):
    c = pl.program_id(0)

    def fetch_rows(cls, b):
        for i in range(P):
            pltpu.make_async_copy(
                xany_ref.at[pl.ds(idx_ref[cls, i], 1)],
                rows_ref.at[b, pl.ds(i, 1)],
                sems.at[b]).start()

    def wait_rows(cls, b):
        for i in range(P):
            pltpu.make_async_copy(
                xany_ref.at[pl.ds(idx_ref[cls, i], 1)],
                rows_ref.at[b, pl.ds(i, 1)],
                sems.at[b]).wait()

    @pl.when(c == 0)
    def _():
        fetch_rows(0, 0)
        pc = lax.dot_general(x_ref[...], Wc_ref[...],
                             (((1,), (1,)), ((), ())),
                             preferred_element_type=jnp.float32)
        pclass_ref[...] = pc + bc_ref[...]

    @pl.when(c + 1 < pl.num_programs(0))
    def _():
        fetch_rows(c + 1, (c + 1) % 2)

    wait_rows(c, c % 2)
    d = rows_ref[c % 2, :P, :].astype(jnp.bfloat16)
    w = Ww_ref[0].astype(jnp.bfloat16)
    pw = lax.dot_general(d, w, (((1,), (1,)), ((), ())),
                         preferred_element_type=jnp.float32)
    pwords_ref[0] = pw + bw_ref[0]


def _tc_decode_dma(idx, x, Wc, bc2, Ww, bw3, nsteps=NCLS):
    grid_spec = pltpu.PrefetchScalarGridSpec(
        num_scalar_prefetch=1,
        grid=(nsteps,),
        in_specs=[
            pl.BlockSpec((T, NHID), lambda c, i_: (0, 0)),
            pl.BlockSpec(memory_space=pl.ANY),
            pl.BlockSpec((NCLS, NHID), lambda c, i_: (0, 0)),
            pl.BlockSpec((1, NCLS), lambda c, i_: (0, 0)),
            pl.BlockSpec((1, CHUNK, NHID), lambda c, i_: (c, 0, 0)),
            pl.BlockSpec((1, 1, CHUNK), lambda c, i_: (c, 0, 0)),
        ],
        out_specs=[
            pl.BlockSpec((T, NCLS), lambda c, i_: (0, 0)),
            pl.BlockSpec((1, P, CHUNK), lambda c, i_: (c, 0, 0)),
        ],
        scratch_shapes=[
            pltpu.VMEM((2, P, NHID), jnp.float32),
            pltpu.SemaphoreType.DMA((2,)),
        ],
    )
    return pl.pallas_call(
        _tcg_body,
        grid_spec=grid_spec,
        out_shape=[
            jax.ShapeDtypeStruct((T, NCLS), jnp.float32),
            jax.ShapeDtypeStruct((NCLS, P, CHUNK), jnp.float32),
        ],
    )(idx, x, x, Wc, bc2, Ww, bw3)


def _tc1_body(idx_ref, x_ref, Wc_ref, bc_ref, Ww_ref, bw_ref,
              pclass_ref, pwords_ref):
    c = pl.program_id(0)

    @pl.when(c == 0)
    def _():
        pc = lax.dot_general(x_ref[...], Wc_ref[...],
                             (((1,), (1,)), ((), ())),
                             preferred_element_type=jnp.float32)
        pclass_ref[...] = pc + bc_ref[...]

    rows = [x_ref[pl.ds(idx_ref[c, i], 1), :] for i in range(P)]
    d = jnp.concatenate(rows, axis=0)
    w = Ww_ref[0]
    pw = lax.dot_general(d, w, (((1,), (1,)), ((), ())),
                         preferred_element_type=jnp.float32)
    pwords_ref[0] = pw + bw_ref[0]


def _tc1(idx_lo, x, Wc, bc2, Ww, bw3):
    grid_spec = pltpu.PrefetchScalarGridSpec(
        num_scalar_prefetch=1,
        grid=(H,),
        in_specs=[
            pl.BlockSpec((T, NHID), lambda c, i_: (0, 0)),
            pl.BlockSpec((NCLS, NHID), lambda c, i_: (0, 0)),
            pl.BlockSpec((1, NCLS), lambda c, i_: (0, 0)),
            pl.BlockSpec((1, CHUNK, NHID), lambda c, i_: (c, 0, 0)),
            pl.BlockSpec((1, 1, CHUNK), lambda c, i_: (c, 0, 0)),
        ],
        out_specs=[
            pl.BlockSpec((T, NCLS), lambda c, i_: (0, 0)),
            pl.BlockSpec((1, P, CHUNK), lambda c, i_: (c, 0, 0)),
        ],
    )
    return pl.pallas_call(
        _tc1_body,
        grid_spec=grid_spec,
        out_shape=[
            jax.ShapeDtypeStruct((T, NCLS), jnp.float32),
            jax.ShapeDtypeStruct((NCLS, P, CHUNK), jnp.float32),
        ],
    )(idx_lo, x, Wc, bc2, Ww, bw3)


def _tc2_body(d_ref, Ww_ref, bw_ref, pw_in_ref, pwords_ref):
    d = d_ref[0, :P, :]
    w = Ww_ref[0]
    pw = lax.dot_general(d, w, (((1,), (1,)), ((), ())),
                         preferred_element_type=jnp.float32)
    pwords_ref[0] = pw + bw_ref[0]


def _tc2(d_hi, Ww, bw3, pw_partial):
    return pl.pallas_call(
        _tc2_body,
        grid=(NCLS - H,),
        in_specs=[
            pl.BlockSpec((1, POUT, NHID), lambda c: (c, 0, 0)),
            pl.BlockSpec((1, CHUNK, NHID), lambda c: (c + H, 0, 0)),
            pl.BlockSpec((1, 1, CHUNK), lambda c: (c + H, 0, 0)),
            pl.BlockSpec(memory_space=pl.ANY),
        ],
        out_specs=pl.BlockSpec((1, P, CHUNK), lambda c: (c + H, 0, 0)),
        out_shape=jax.ShapeDtypeStruct((NCLS, P, CHUNK), jnp.float32),
        input_output_aliases={3: 0},
    )(d_hi, Ww, bw3, pw_partial)


def kernel(x, within_batch_idx, Wc, bc, Ww, bw):
    idx32 = within_batch_idx.astype(jnp.int32)
    idx_hi = idx32[H:]
    idx_hi = jnp.pad(idx_hi, ((0, SLOTS * NW - NSC), (0, PPAD - P)))
    idx3 = idx_hi.reshape(SLOTS, NW, PPAD)
    bc2 = bc.reshape(1, NCLS)
    bw3 = bw.reshape(NCLS, 1, CHUNK)

    del idx3
    p_class, p_words = _tc_decode_dma(idx32, x, Wc, bc2, Ww, bw3)
    return (p_class, p_words)


# speedup vs baseline: 1.0008x; 1.0008x over previous
import jax
import jax.numpy as jnp
from jax import lax
from jax.experimental import pallas as pl
from jax.experimental.pallas import tpu as pltpu

T = 2048
NHID = 1024
NCLS = 100
CHUNK = 1000
P = 20


def _pclass_body(x_ref, Wc_ref, bc_ref, pclass_ref):
    pc = lax.dot_general(x_ref[...], Wc_ref[...],
                         (((1,), (1,)), ((), ())),
                         preferred_element_type=jnp.float32)
    pclass_ref[...] = pc + bc_ref[...]


def _pclass(x, Wc, bc2):
    return pl.pallas_call(
        _pclass_body,
        out_shape=jax.ShapeDtypeStruct((T, NCLS), jnp.float32),
    )(x, Wc, bc2)


def _tcg_body(idx_ref, x_ref, Ww_ref, bw_ref, pwords_ref, rows_ref, sems):
    c = pl.program_id(0)

    def fetch_rows(cls, b):
        for i in range(P):
            pltpu.make_async_copy(
                x_ref.at[pl.ds(idx_ref[cls, i], 1)],
                rows_ref.at[b, pl.ds(i, 1)],
                sems.at[b]).start()

    def wait_rows(cls, b):
        for i in range(P):
            pltpu.make_async_copy(
                x_ref.at[pl.ds(idx_ref[cls, i], 1)],
                rows_ref.at[b, pl.ds(i, 1)],
                sems.at[b]).wait()

    @pl.when(c == 0)
    def _():
        fetch_rows(0, 0)

    @pl.when(c + 1 < pl.num_programs(0))
    def _():
        fetch_rows(c + 1, (c + 1) % 2)

    wait_rows(c, c % 2)
    d = rows_ref[c % 2, :P, :]
    w = Ww_ref[0]
    pw = lax.dot_general(d, w, (((1,), (1,)), ((), ())),
                         preferred_element_type=jnp.float32)
    pwords_ref[0] = pw + bw_ref[0]


def _pwords(idx, x, Ww, bw3):
    grid_spec = pltpu.PrefetchScalarGridSpec(
        num_scalar_prefetch=1,
        grid=(NCLS,),
        in_specs=[
            pl.BlockSpec(memory_space=pl.ANY),
            pl.BlockSpec((1, CHUNK, NHID), lambda c, i_: (c, 0, 0)),
            pl.BlockSpec((1, 1, CHUNK), lambda c, i_: (c, 0, 0)),
        ],
        out_specs=pl.BlockSpec((1, P, CHUNK), lambda c, i_: (c, 0, 0)),
        scratch_shapes=[
            pltpu.VMEM((2, P, NHID), jnp.float32),
            pltpu.SemaphoreType.DMA((2,)),
        ],
    )
    return pl.pallas_call(
        _tcg_body,
        grid_spec=grid_spec,
        out_shape=jax.ShapeDtypeStruct((NCLS, P, CHUNK), jnp.float32),
    )(idx, x, Ww, bw3)


def kernel(x, within_batch_idx, Wc, bc, Ww, bw):
    idx32 = within_batch_idx.astype(jnp.int32)
    p_class = _pclass(x, Wc, bc.reshape(1, NCLS))
    p_words = _pwords(idx32, x, Ww, bw.reshape(NCLS, 1, CHUNK))
    return (p_class, p_words)


# speedup vs baseline: 1.1799x; 1.1790x over previous
import jax
import jax.numpy as jnp
from jax import lax
from jax.experimental import pallas as pl
from jax.experimental.pallas import tpu as pltpu

T = 2048
NHID = 1024
NCLS = 100
CHUNK = 1000
P = 20
G = 2
NSTEP = NCLS // G


def _tcg_body(idx_ref, x_ref, xany_ref, Wc_ref, bc_ref, Ww_ref, bw_ref,
              pclass_ref, pwords_ref, rows_ref, sems):
    c = pl.program_id(0)

    def fetch_rows(step, b):
        for j in range(G):
            for i in range(P):
                pltpu.make_async_copy(
                    xany_ref.at[pl.ds(idx_ref[step * G + j, i], 1)],
                    rows_ref.at[b, pl.ds(j * P + i, 1)],
                    sems.at[b]).start()

    def wait_rows(step, b):
        for j in range(G):
            for i in range(P):
                pltpu.make_async_copy(
                    xany_ref.at[pl.ds(idx_ref[step * G + j, i], 1)],
                    rows_ref.at[b, pl.ds(j * P + i, 1)],
                    sems.at[b]).wait()

    @pl.when(c == 0)
    def _():
        fetch_rows(0, 0)
        pc = lax.dot_general(x_ref[...], Wc_ref[...],
                             (((1,), (1,)), ((), ())),
                             preferred_element_type=jnp.float32)
        pclass_ref[...] = pc + bc_ref[...]

    @pl.when(c + 1 < pl.num_programs(0))
    def _():
        fetch_rows(c + 1, (c + 1) % 2)

    wait_rows(c, c % 2)
    for j in range(G):
        d = rows_ref[c % 2, j * P:(j + 1) * P, :]
        w = Ww_ref[j]
        pw = lax.dot_general(d, w, (((1,), (1,)), ((), ())),
                             preferred_element_type=jnp.float32)
        pwords_ref[j] = pw + bw_ref[j]


def _decode(idx, x, Wc, bc2, Ww, bw3):
    grid_spec = pltpu.PrefetchScalarGridSpec(
        num_scalar_prefetch=1,
        grid=(NSTEP,),
        in_specs=[
            pl.BlockSpec((T, NHID), lambda c, i_: (0, 0)),
            pl.BlockSpec(memory_space=pl.ANY),
            pl.BlockSpec((NCLS, NHID), lambda c, i_: (0, 0)),
            pl.BlockSpec((1, NCLS), lambda c, i_: (0, 0)),
            pl.BlockSpec((G, CHUNK, NHID), lambda c, i_: (c, 0, 0)),
            pl.BlockSpec((G, 1, CHUNK), lambda c, i_: (c, 0, 0)),
        ],
        out_specs=[
            pl.BlockSpec((T, NCLS), lambda c, i_: (0, 0)),
            pl.BlockSpec((G, P, CHUNK), lambda c, i_: (c, 0, 0)),
        ],
        scratch_shapes=[
            pltpu.VMEM((2, G * P, NHID), jnp.float32),
            pltpu.SemaphoreType.DMA((2,)),
        ],
    )
    return pl.pallas_call(
        _tcg_body,
        grid_spec=grid_spec,
        out_shape=[
            jax.ShapeDtypeStruct((T, NCLS), jnp.float32),
            jax.ShapeDtypeStruct((NCLS, P, CHUNK), jnp.float32),
        ],
    )(idx, x, x, Wc, bc2, Ww, bw3)


def kernel(x, within_batch_idx, Wc, bc, Ww, bw):
    idx32 = within_batch_idx.astype(jnp.int32)
    p_class, p_words = _decode(idx32, x, Wc, bc.reshape(1, NCLS),
                               Ww, bw.reshape(NCLS, 1, CHUNK))
    return (p_class, p_words)
